# Initial kernel scaffold; baseline (speedup 1.0000x reference)
import jax
import jax.numpy as jnp
from jax.experimental import pallas as pl


def kernel(xyz, xyz1, f1, xyz2, f2, xyz3, f3, xyz4, f4, params):
    raise NotImplementedError("write your pallas kernel here")


# speedup vs baseline: 28.0035x; 28.0035x over previous
import functools

import jax
import jax.numpy as jnp
from jax import lax
from jax.experimental import pallas as pl
from jax.experimental.pallas import tpu as pltpu
from jax.experimental.pallas import tpu_sc as plsc

_NC = 2
_NS = 16
_NW = _NC * _NS


def _knn_body(x1_ref, x2_ref, idx_ref, w_ref, *, S, Tb):
    b = pl.program_id(0)
    x1 = x1_ref[0]
    x2 = x2_ref[0]
    n1 = jnp.sum(x1 * x1, axis=0, keepdims=True)
    n2 = jnp.sum(x2 * x2, axis=1, keepdims=True)
    xx = jnp.dot(x2, x1, preferred_element_type=jnp.float32)
    d = (n1 - 2.0 * xx) + n2
    iota = lax.broadcasted_iota(jnp.int32, (S, Tb), 0)
    ms, ids = [], []
    for k in range(3):
        m = jnp.min(d, axis=0, keepdims=True)
        i = jnp.min(jnp.where(d == m, iota, jnp.int32(S)), axis=0,
                    keepdims=True)
        ms.append(m)
        ids.append(i)
        if k < 2:
            d = jnp.where(iota == i, jnp.float32(1e30), d)
    r = [1.0 / (m + 1e-8) for m in ms]
    norm = r[0] + r[1] + r[2]
    for k in range(3):
        idx_ref[0, 0, k, :] = (ids[k] + b * S)[0]
        w_ref[0, 0, k, :] = (r[k] / norm)[0]


def _knn_topk(x1p, x2t, Tb):
    B, _, T = x1p.shape
    S = x2t.shape[1]
    nTb = T // Tb
    idx, w = pl.pallas_call(
        functools.partial(_knn_body, S=S, Tb=Tb),
        grid=(B, nTb),
        in_specs=[
            pl.BlockSpec((1, 8, Tb), lambda b, j: (b, 0, j)),
            pl.BlockSpec((1, S, 8), lambda b, j: (b, 0, 0)),
        ],
        out_specs=[
            pl.BlockSpec((1, 1, 3, Tb), lambda b, j: (b, j, 0, 0)),
            pl.BlockSpec((1, 1, 3, Tb), lambda b, j: (b, j, 0, 0)),
        ],
        out_shape=[
            jax.ShapeDtypeStruct((B, nTb, 3, Tb), jnp.int32),
            jax.ShapeDtypeStruct((B, nTb, 3, Tb), jnp.float32),
        ],
    )(x1p, x2t)
    idx_flat = idx.transpose(2, 0, 1, 3).reshape(3 * B * T)
    w_rows = w.transpose(0, 1, 3, 2).reshape(B * T, 3)
    return idx_flat, w_rows


def _sc_gather(table, idx):
    V, D = table.shape
    rows = idx.shape[0]
    b_per_w = rows // _NW
    chunk = b_per_w
    while chunk * D * 4 > 420_000:
        chunk //= 2
    nchunks = b_per_w // chunk
    mesh = plsc.VectorSubcoreMesh(core_axis_name="c", subcore_axis_name="s")

    @functools.partial(
        pl.kernel,
        out_type=jax.ShapeDtypeStruct((rows, D), jnp.float32),
        mesh=mesh,
        scratch_types=[
            pltpu.VMEM((chunk,), jnp.int32),
            pltpu.VMEM((chunk, D), jnp.float32),
            pltpu.SemaphoreType.DMA,
        ],
    )
    def gk(table_hbm, idx_hbm, out_hbm, idx_v, rows_v, sem):
        wid = lax.axis_index("s") * _NC + lax.axis_index("c")
        for j in range(nchunks):
            base = wid * b_per_w + j * chunk
            pltpu.sync_copy(idx_hbm.at[pl.ds(base, chunk)], idx_v)
            pltpu.async_copy(table_hbm.at[idx_v], rows_v, sem).wait()
            pltpu.sync_copy(rows_v, out_hbm.at[pl.ds(base, chunk)])

    return gk(table, idx)


def _norm_relu(h, gamma, beta):
    mean = jnp.mean(h, axis=0, keepdims=True)
    c = h - mean
    var = jnp.mean(c * c, axis=0, keepdims=True)
    return jnp.maximum(gamma * c / jnp.sqrt(var + 1e-5) + beta, 0.0)


def _dot(a, b):
    return jnp.dot(a, b, preferred_element_type=jnp.float32)


def _fp_mlp(g0, g1, g2, w, p1, lay1, lay2, head=None):
    R = g0.shape[0]
    W1pT, W1iT, b1, ga1, be1 = lay1
    _, W2T, b2, ga2, be2 = lay2
    ops = [g0, g1, g2, w]
    if p1 is not None:
        ops.append(p1)
    ops += [W1iT, b1, ga1, be1, W2T, b2, ga2, be2]
    if p1 is not None:
        ops.append(W1pT)
    if head is not None:
        ops += list(head)
    out_dim = head[4].shape[1] if head is not None else W2T.shape[1]

    def body(*refs):
        g0r, g1r, g2r, wr = refs[0], refs[1], refs[2], refs[3]
        i = 4
        p1r = None
        if p1 is not None:
            p1r = refs[i]
            i += 1
        W1iTr, b1r, ga1r, be1r, W2Tr, b2r, ga2r, be2r = refs[i:i + 8]
        i += 8
        W1pTr = None
        if p1 is not None:
            W1pTr = refs[i]
            i += 1
        if head is not None:
            c1Tr, cb1r, hg, hb, c2Tr, cb2r = refs[i:i + 6]
            i += 6
        out_ref = refs[-1]
        interp = (g0r[...] * wr[:, 0:1] + g1r[...] * wr[:, 1:2]
                  + g2r[...] * wr[:, 2:3])
        h = _dot(interp, W1iTr[...])
        if p1r is not None:
            h = h + _dot(p1r[...], W1pTr[...])
        h = _norm_relu(h + b1r[...], ga1r[...], be1r[...])
        h = _norm_relu(_dot(h, W2Tr[...]) + b2r[...], ga2r[...], be2r[...])
        if head is not None:
            x = _norm_relu(_dot(h, c1Tr[...]) + cb1r[...], hg[...], hb[...])
            x = _dot(x, c2Tr[...]) + cb2r[...]
            x = x - jnp.max(x, axis=1, keepdims=True)
            x = x - jnp.log(jnp.sum(jnp.exp(x), axis=1, keepdims=True))
            out_ref[...] = x
        else:
            out_ref[...] = h

    return pl.pallas_call(
        body,
        out_shape=jax.ShapeDtypeStruct((R, out_dim), jnp.float32),
    )(*ops)


def _rows(x):
    return x.transpose(0, 2, 1).reshape(x.shape[0] * x.shape[2], x.shape[1])


def _pad_t(x):
    B, _, N = x.shape
    return jnp.concatenate([x, jnp.zeros((B, 5, N), x.dtype)], axis=1)


def _layer_params(layer, split):
    W, b, g, be = layer
    row = lambda v: v.reshape(1, -1)
    if split:
        return (W[:, :split].T, W[:, split:].T, row(b), row(g), row(be))
    return (None, W.T, row(b), row(g), row(be))


def kernel(xyz, xyz1, f1, xyz2, f2, xyz3, f3, xyz4, f4, params):
    B = xyz.shape[0]

    xt = {k: _pad_t(v) for k, v in
          dict(x0=xyz, x1=xyz1, x2=xyz2, x3=xyz3, x4=xyz4).items()}
    xs = {k: v.transpose(0, 2, 1) for k, v in xt.items()}

    p = params
    head = (p['conv1_w'].T, p['conv1_b'].reshape(1, -1),
            p['bn1_g'].reshape(1, -1), p['bn1_b'].reshape(1, -1),
            p['conv2_w'].T, p['conv2_b'].reshape(1, -1))

    levels = [
        ('x3', 'x4', f3, 'fp1', 128, 128),
        ('x2', 'x3', f2, 'fp2', 64, 512),
        ('x1', 'x2', f1, 'fp3', 64, 512),
        ('x0', 'x1', None, 'fp4', 0, 512),
    ]

    table = _rows(f4)
    out = None
    for (tk, sk, skip, pk, c1, Tb) in levels:
        last = pk == 'fp4'
        T = xt[tk].shape[2]
        R = B * T
        idx_flat, w_rows = _knn_topk(xt[tk], xs[sk], Tb)
        g = _sc_gather(table, idx_flat)
        lay1 = _layer_params(p[pk][0], c1)
        lay2 = _layer_params(p[pk][1], 0)
        res = _fp_mlp(g[0:R], g[R:2 * R], g[2 * R:3 * R], w_rows,
                      _rows(skip) if skip is not None else None,
                      lay1, lay2, head=head if last else None)
        if last:
            out = res.reshape(B, T, res.shape[1])
        else:
            table = res

    return (out, f4)
